# baseline (speedup 1.0000x reference)
import dataclasses
import functools

import jax
import jax.numpy as jnp
from jax import lax
from jax.experimental import pallas as pl
from jax.experimental.pallas import tpu as pltpu
from jax.experimental.pallas import tpu_sc as plsc

B, T, V, C = 16384, 200, 10, 3
L = 16
TH = T // 8
BHQ = B // 128
BH = 2


def kernel(idx, weight):
    idx4 = idx.T.reshape(TH, 8, BHQ, 128).transpose(0, 2, 1, 3)
    mesh = plsc.VectorSubcoreMesh(core_axis_name="c", subcore_axis_name="s")

    @functools.partial(
        pl.kernel,
        out_type=jax.ShapeDtypeStruct((V, TH, BHQ, 8, 128), jnp.float32),
        mesh=mesh,
        scratch_types=[pltpu.VMEM((128,), jnp.int32),
                       pltpu.VMEM((V, C), jnp.float32)],
        compiler_params=dataclasses.replace(
            pltpu.CompilerParams(), needs_layout_passes=False),
    )
    def sc_kern(idx_hbm, w_hbm, out_hbm, table_vmem, w_vmem):
        pltpu.sync_copy(w_hbm, w_vmem)

        def _rne_bf16_bits(x):
            bits = lax.bitcast_convert_type(x, jnp.int32)
            rnd = (lax.shift_right_logical(bits, 16) & 1) + 0x7FFF
            return lax.shift_right_logical(bits + rnd, 16)

        for v in range(5):
            s = lax.iota(jnp.int32, L) + (16 * v)
            k = s // 8
            jj = jnp.minimum(s - k * 8, 4)
            j0 = 2 * jj
            j1 = jnp.minimum(j0 + 1, V - 1)
            g0 = jnp.zeros((L,), jnp.float32)
            g1 = jnp.zeros((L,), jnp.float32)
            for c in range(C):
                cc = jnp.full((L,), c, jnp.int32)
                wk = plsc.load_gather(w_vmem, [k, cc])
                g0 = g0 + wk * plsc.load_gather(w_vmem, [j0, cc])
                g1 = g1 + wk * plsc.load_gather(w_vmem, [j1, cc])
            word = (_rne_bf16_bits(g0) & 0xFFFF) | lax.shift_left(
                _rne_bf16_bits(g1), 16)
            table_vmem[pl.ds(16 * v, L)] = word

        def body(idx_vmem, out_vmem):
            @pl.loop(0, BH)
            def _(bh):
                for tl in range(8):
                    wbases = []
                    for cc in range(8):
                        idxv = idx_vmem.at[0, bh, tl, pl.ds(16 * cc, L)][...]
                        wbases.append(idxv * 8)
                    for cc in range(8):
                        sl = pl.ds(16 * cc, L)
                        words = [plsc.load_gather(table_vmem, [wbases[cc] + jj])
                                 for jj in range(V // 2)]
                        for jj in range(V // 2):
                            wv = words[jj]
                            lo = lax.bitcast_convert_type(
                                lax.shift_left(wv, 16), jnp.float32)
                            hi = lax.bitcast_convert_type(
                                wv & jnp.int32(-65536), jnp.float32)
                            out_vmem.at[2 * jj, 0, bh, tl, sl][...] = lo
                            out_vmem.at[2 * jj + 1, 0, bh, tl, sl][...] = hi

        pltpu.emit_pipeline(
            body,
            grid=(TH, BHQ // BH),
            in_specs=[pl.BlockSpec((1, BH, 8, 128),
                                   index_map=lambda th, s: (th, s, 0, 0))],
            out_specs=[pl.BlockSpec((V, 1, BH, 8, 128),
                                    index_map=lambda th, s: (0, th, s, 0, 0))],
            core_axis_name=("c", "s"),
            dimension_semantics=(pltpu.PARALLEL, pltpu.PARALLEL),
        )(idx_hbm, out_hbm)

    out5 = sc_kern(idx4, weight)
    return out5.transpose(2, 4, 1, 3, 0).reshape(B, T, V)


# speedup vs baseline: 1.2309x; 1.2309x over previous
import dataclasses
import functools

import jax
import jax.numpy as jnp
from jax import lax
from jax.experimental import pallas as pl
from jax.experimental.pallas import tpu as pltpu
from jax.experimental.pallas import tpu_sc as plsc

B, T, V, C = 16384, 200, 10, 3
L = 16
TH = T // 8
BHQ = B // 128
BH = 2


def kernel(idx, weight):
    idx4 = idx.T.reshape(TH, 8, BHQ, 128).transpose(0, 2, 1, 3)
    mesh = plsc.VectorSubcoreMesh(core_axis_name="c", subcore_axis_name="s")

    @functools.partial(
        pl.kernel,
        out_type=jax.ShapeDtypeStruct((V, TH, BHQ, 8, 128), jnp.float32),
        mesh=mesh,
        scratch_types=[pltpu.VMEM((128,), jnp.float32),
                       pltpu.VMEM((V, C), jnp.float32)],
        compiler_params=dataclasses.replace(
            pltpu.CompilerParams(), needs_layout_passes=False),
    )
    def sc_kern(idx_hbm, w_hbm, out_hbm, table_vmem, w_vmem):
        pltpu.sync_copy(w_hbm, w_vmem)
        for v in range(7):
            e = lax.iota(jnp.int32, L) + (16 * v)
            k = jnp.minimum(e // V, V - 1)
            j2 = jnp.minimum(e - (e // V) * V, V - 1)
            acc = jnp.zeros((L,), jnp.float32)
            for c in range(C):
                cc = jnp.full((L,), c, jnp.int32)
                acc = acc + (plsc.load_gather(w_vmem, [k, cc]) *
                             plsc.load_gather(w_vmem, [j2, cc]))
            table_vmem[pl.ds(16 * v, L)] = acc

        full = lax.iota(jnp.int32, L) >= 0

        def body(idx_vmem, out_vmem):
            @pl.loop(0, BH)
            def _(bh):
                for tl in range(8):
                    wbases = []
                    for cc in range(8):
                        idxv = idx_vmem.at[0, bh, tl, pl.ds(16 * cc, L)][...]
                        wbases.append(idxv * V)
                    for cc in range(8):
                        sl = pl.ds(16 * cc, L)
                        vals = [plsc.load_gather(table_vmem, [wbases[cc] + j])
                                for j in range(V)]
                        for j in range(V):
                            plsc.store_compressed(
                                out_vmem.at[j, 0, bh, tl, sl], vals[j],
                                mask=full)

        pltpu.emit_pipeline(
            body,
            grid=(TH, BHQ // BH),
            in_specs=[pl.BlockSpec((1, BH, 8, 128),
                                   index_map=lambda th, s: (th, s, 0, 0))],
            out_specs=[pl.BlockSpec((V, 1, BH, 8, 128),
                                    index_map=lambda th, s: (0, th, s, 0, 0))],
            core_axis_name=("c", "s"),
            dimension_semantics=(pltpu.PARALLEL, pltpu.PARALLEL),
        )(idx_hbm, out_hbm)

    out5 = sc_kern(idx4, weight)
    return out5.transpose(2, 4, 1, 3, 0).reshape(B, T, V)
